# baseline (speedup 1.0000x reference)
import jax
import jax.numpy as jnp
from jax.experimental import pallas as pl
from jax.experimental.pallas import tpu as pltpu


_NCHUNK = 32


def _body(x_hbm, hp_ref, out_ref, x_vmem, sems):
    nv, fv = x_vmem.shape
    rows = nv // _NCHUNK

    def copy(c):
        return pltpu.make_async_copy(
            x_hbm.at[pl.ds(c * rows, rows), :],
            x_vmem.at[pl.ds(c * rows, rows), :],
            sems.at[c],
        )

    for c in range(_NCHUNK):
        copy(c).start()

    k = hp_ref[0, :]
    m = hp_ref[1, :]
    km = k * m
    a = 0.5 * k
    const = 0.5 * jnp.sum(km * m)
    ones = jnp.ones((1, fv), dtype=jnp.float32)
    for c in range(_NCHUNK):
        copy(c).wait()
        x = x_vmem[pl.ds(c * rows, rows), :]
        t = x * (a[None, :] * x - km[None, :])
        tt = t.T
        e = jax.lax.dot_general(
            ones, tt, (((1,), (0,)), ((), ())),
            preferred_element_type=jnp.float32,
        )
        out_ref[:, pl.ds(c * rows, rows)] = e + const


def kernel(in_feat, harmonic_parameters):
    n, f = in_feat.shape
    out = pl.pallas_call(
        _body,
        in_specs=[
            pl.BlockSpec(memory_space=pltpu.MemorySpace.HBM),
            pl.BlockSpec((2, f), lambda: (0, 0)),
        ],
        out_specs=pl.BlockSpec((1, n), lambda: (0, 0)),
        out_shape=jax.ShapeDtypeStruct((1, n), jnp.float32),
        scratch_shapes=[
            pltpu.VMEM((n, f), jnp.float32),
            pltpu.SemaphoreType.DMA((_NCHUNK,)),
        ],
        grid=(),
    )(in_feat, harmonic_parameters)
    return out.reshape(n, 1)


# speedup vs baseline: 1.2187x; 1.2187x over previous
import jax
import jax.numpy as jnp
from jax.experimental import pallas as pl
from jax.experimental.pallas import tpu as pltpu


_NCHUNK = 8


def _body(x_hbm, hp_ref, out_ref, x_vmem, sems):
    nv, fv = x_vmem.shape
    rows = nv // _NCHUNK

    def copy(c):
        return pltpu.make_async_copy(
            x_hbm.at[pl.ds(c * rows, rows), :],
            x_vmem.at[pl.ds(c * rows, rows), :],
            sems.at[c],
        )

    for c in range(_NCHUNK):
        copy(c).start()

    k = hp_ref[0, :]
    m = hp_ref[1, :]
    km = k * m
    a = 0.5 * k
    const = 0.5 * jnp.sum(km * m)
    ones = jnp.ones((1, fv), dtype=jnp.float32)
    for c in range(_NCHUNK):
        copy(c).wait()
        x = x_vmem[pl.ds(c * rows, rows), :]
        t = x * (a[None, :] * x - km[None, :])
        tt = t.T
        e = jax.lax.dot_general(
            ones, tt, (((1,), (0,)), ((), ())),
            preferred_element_type=jnp.float32,
        )
        out_ref[:, pl.ds(c * rows, rows)] = e + const


def kernel(in_feat, harmonic_parameters):
    n, f = in_feat.shape
    out = pl.pallas_call(
        _body,
        in_specs=[
            pl.BlockSpec(memory_space=pltpu.MemorySpace.HBM),
            pl.BlockSpec((2, f), lambda: (0, 0)),
        ],
        out_specs=pl.BlockSpec((1, n), lambda: (0, 0)),
        out_shape=jax.ShapeDtypeStruct((1, n), jnp.float32),
        scratch_shapes=[
            pltpu.VMEM((n, f), jnp.float32),
            pltpu.SemaphoreType.DMA((_NCHUNK,)),
        ],
        grid=(),
    )(in_feat, harmonic_parameters)
    return out.reshape(n, 1)


# speedup vs baseline: 1.2202x; 1.0012x over previous
import jax
import jax.numpy as jnp
from jax.experimental import pallas as pl
from jax.experimental.pallas import tpu as pltpu


_NCHUNK = 4


def _body(x_hbm, hp_ref, out_ref, x_vmem, sems):
    nv, fv = x_vmem.shape
    rows = nv // _NCHUNK

    def copy(c):
        return pltpu.make_async_copy(
            x_hbm.at[pl.ds(c * rows, rows), :],
            x_vmem.at[pl.ds(c * rows, rows), :],
            sems.at[c],
        )

    for c in range(_NCHUNK):
        copy(c).start()

    k = hp_ref[0, :]
    m = hp_ref[1, :]
    km = k * m
    a = 0.5 * k
    const = 0.5 * jnp.sum(km * m)
    ones = jnp.ones((1, fv), dtype=jnp.float32)
    for c in range(_NCHUNK):
        copy(c).wait()
        x = x_vmem[pl.ds(c * rows, rows), :]
        t = x * (a[None, :] * x - km[None, :])
        tt = t.T
        e = jax.lax.dot_general(
            ones, tt, (((1,), (0,)), ((), ())),
            preferred_element_type=jnp.float32,
        )
        out_ref[:, pl.ds(c * rows, rows)] = e + const


def kernel(in_feat, harmonic_parameters):
    n, f = in_feat.shape
    out = pl.pallas_call(
        _body,
        in_specs=[
            pl.BlockSpec(memory_space=pltpu.MemorySpace.HBM),
            pl.BlockSpec((2, f), lambda: (0, 0)),
        ],
        out_specs=pl.BlockSpec((1, n), lambda: (0, 0)),
        out_shape=jax.ShapeDtypeStruct((1, n), jnp.float32),
        scratch_shapes=[
            pltpu.VMEM((n, f), jnp.float32),
            pltpu.SemaphoreType.DMA((_NCHUNK,)),
        ],
        grid=(),
    )(in_feat, harmonic_parameters)
    return out.reshape(n, 1)
